# baseline (speedup 1.0000x reference)
import functools

import jax
import jax.numpy as jnp
from jax import lax
from jax.experimental import pallas as pl
from jax.experimental.pallas import tpu as pltpu

B, N, D = 128, 256, 64
M = B * N


def _vq_body(z_ref, e_ref, zq_ref, idx_ref, loss_ref, perp_ref,
             cnt_ref, acc_ref, *, bm, k, grid):
    i = pl.program_id(0)

    z = z_ref[...]
    e = e_ref[...]

    z2 = jnp.sum(z * z, axis=1, keepdims=True)
    e2 = lax.dot_general(jnp.ones((1, D), jnp.float32), e * e,
                         (((1,), (1,)), ((), ())),
                         precision=lax.Precision.HIGHEST)
    dot = lax.dot_general(z, e, (((1,), (1,)), ((), ())),
                          precision=lax.Precision.DEFAULT,
                          preferred_element_type=jnp.float32)
    dist = (z2 - 2.0 * dot) + e2

    idx = jnp.argmin(dist, axis=1).astype(jnp.int32)

    iota = lax.broadcasted_iota(jnp.int32, (bm, k), 1)
    onehot = (iota == idx[:, None]).astype(jnp.float32)
    zq = lax.dot_general(onehot, e, (((1,), (0,)), ((), ())),
                         precision=lax.Precision.HIGHEST)

    zq_ref[...] = zq
    idx_ref[...] = idx.reshape(1, 1, bm)

    @pl.when(i == 0)
    def _init():
        cnt_ref[...] = jnp.zeros_like(cnt_ref)
        acc_ref[0, 0] = 0.0

    cnt_ref[...] += jnp.sum(onehot, axis=0, keepdims=True)
    diff = zq - z
    acc_ref[0, 0] += jnp.sum(diff * diff)

    @pl.when(i == grid - 1)
    def _fin():
        total = acc_ref[0, 0]
        loss_ref[...] = jnp.full((1, 1), 1.25 * total / float(M * D),
                                 jnp.float32)
        p = cnt_ref[...] * (1.0 / float(M))
        ent = -jnp.sum(p * jnp.log(p + 1e-10))
        perp_ref[...] = jnp.full((1, 1), jnp.exp(ent), jnp.float32)


@functools.partial(jax.jit, static_argnums=(2,))
def _vq_one(z, embed, bm):
    k = embed.shape[0]
    flat = z.reshape(M, D)
    grid = M // bm

    zq, idx3, loss, perp = pl.pallas_call(
        functools.partial(_vq_body, bm=bm, k=k, grid=grid),
        grid=(grid,),
        in_specs=[
            pl.BlockSpec((bm, D), lambda i: (i, 0)),
            pl.BlockSpec((k, D), lambda i: (0, 0)),
        ],
        out_specs=[
            pl.BlockSpec((bm, D), lambda i: (i, 0)),
            pl.BlockSpec((1, 1, bm), lambda i: (i, 0, 0)),
            pl.BlockSpec((1, 1), lambda i: (0, 0)),
            pl.BlockSpec((1, 1), lambda i: (0, 0)),
        ],
        out_shape=[
            jax.ShapeDtypeStruct((M, D), jnp.float32),
            jax.ShapeDtypeStruct((grid, 1, bm), jnp.int32),
            jax.ShapeDtypeStruct((1, 1), jnp.float32),
            jax.ShapeDtypeStruct((1, 1), jnp.float32),
        ],
        scratch_shapes=[
            pltpu.VMEM((1, k), jnp.float32),
            pltpu.SMEM((1, 1), jnp.float32),
        ],
        compiler_params=pltpu.CompilerParams(
            dimension_semantics=("arbitrary",),
        ),
    )(flat, embed)

    z_q = zq.reshape(B, N, D)
    indices = idx3.reshape(B, N)
    return z_q, indices, loss[0, 0], perp[0, 0]


def kernel(z_l, z_h, embed_l, embed_h):
    z_q_l, idx_l, loss_l, perp_l = _vq_one(z_l, embed_l, 512)
    z_q_h, idx_h, loss_h, perp_h = _vq_one(z_h, embed_h, 512)
    return (z_q_l, idx_l, loss_l, perp_l, z_q_h, idx_h, loss_h, perp_h)


# speedup vs baseline: 2.1634x; 2.1634x over previous
import functools

import jax
import jax.numpy as jnp
from jax import lax
from jax.experimental import pallas as pl
from jax.experimental.pallas import tpu as pltpu
from jax.experimental.pallas import tpu_sc as plsc

B, N, D = 128, 256, 64
M = B * N


def _vq_tc_body(z_ref, e_ref, idx_ref, loss_ref,
                e2_ref, acc_ref, *, bm, k, grid):
    i = pl.program_id(0)

    z = z_ref[...]
    e = e_ref[...]

    @pl.when(i == 0)
    def _init():
        e2_ref[...] = jnp.sum(e * e, axis=1, keepdims=True)
        acc_ref[0, 0] = 0.0

    z2 = jnp.sum(z * z, axis=1, keepdims=True)
    z2t = lax.transpose(z2, (1, 0))
    dotT = lax.dot_general(e, z, (((1,), (1,)), ((), ())),
                           precision=lax.Precision.DEFAULT,
                           preferred_element_type=jnp.float32)
    distT = (z2t - 2.0 * dotT) + e2_ref[...]

    idx = jnp.argmin(distT, axis=0).astype(jnp.int32)
    idx_ref[...] = idx.reshape(1, 1, bm)

    minv = jnp.min(distT, axis=0)
    acc_ref[0, 0] += jnp.sum(minv)

    @pl.when(i == grid - 1)
    def _fin():
        total = acc_ref[0, 0]
        loss_ref[...] = jnp.full((1, 1), 1.25 * total / float(M * D),
                                 jnp.float32)


@functools.partial(jax.jit, static_argnums=(2,))
def _vq_tc(z, embed, bm):
    k = embed.shape[0]
    flat = z.reshape(M, D)
    grid = M // bm

    idx3, loss = pl.pallas_call(
        functools.partial(_vq_tc_body, bm=bm, k=k, grid=grid),
        grid=(grid,),
        in_specs=[
            pl.BlockSpec((bm, D), lambda i: (i, 0)),
            pl.BlockSpec((k, D), lambda i: (0, 0)),
        ],
        out_specs=[
            pl.BlockSpec((1, 1, bm), lambda i: (i, 0, 0)),
            pl.BlockSpec((1, 1), lambda i: (0, 0)),
        ],
        out_shape=[
            jax.ShapeDtypeStruct((grid, 1, bm), jnp.int32),
            jax.ShapeDtypeStruct((1, 1), jnp.float32),
        ],
        scratch_shapes=[
            pltpu.VMEM((k, 1), jnp.float32),
            pltpu.SMEM((1, 1), jnp.float32),
        ],
        compiler_params=pltpu.CompilerParams(
            dimension_semantics=("arbitrary",),
        ),
    )(flat, embed)

    return idx3.reshape(M), loss[0, 0]


_CHUNK = 128


def _make_sc_gather_hist(k):
    info = plsc.get_sparse_core_info()
    nc, ns, nl = info.num_cores, info.num_subcores, info.num_lanes
    nw = nc * ns
    b_per_w = M // nw
    n_chunks = b_per_w // _CHUNK
    mesh = plsc.VectorSubcoreMesh(core_axis_name="c", subcore_axis_name="s")

    @functools.partial(
        pl.kernel, mesh=mesh,
        out_type=[
            jax.ShapeDtypeStruct((M, D), jnp.float32),
            jax.ShapeDtypeStruct((nc, k), jnp.float32),
        ],
        scratch_types=[
            pltpu.VMEM((n_chunks, _CHUNK), jnp.int32),
            pltpu.VMEM((b_per_w, D), jnp.float32),
            pltpu.VMEM((_CHUNK,), jnp.float32),
            pltpu.VMEM((_CHUNK,), jnp.float32),
            pltpu.VMEM_SHARED((k,), jnp.float32),
            pltpu.SemaphoreType.DMA,
        ],
        compiler_params=pltpu.CompilerParams(use_tc_tiling_on_sc=False),
    )
    def gather_hist(table_hbm, idx_hbm, zq_hbm, cnt_hbm,
                    idx_v, rows_v, ones_v, zeros_v, shared, sem):
        cid = lax.axis_index("c")
        sid = lax.axis_index("s")
        wid = sid * nc + cid
        pltpu.sync_copy(idx_hbm.at[pl.ds(wid * n_chunks, n_chunks), :], idx_v)

        copies = []
        for j in range(n_chunks):
            copies.append(pltpu.async_copy(
                table_hbm.at[idx_v.at[j]],
                rows_v.at[pl.ds(j * _CHUNK, _CHUNK), :],
                sem))

        for t in range(_CHUNK // nl):
            ones_v[pl.ds(t * nl, nl)] = jnp.full((nl,), 1.0, jnp.float32)

        @pl.when(sid == 0)
        def _zero():
            for t in range(_CHUNK // nl):
                zeros_v[pl.ds(t * nl, nl)] = jnp.zeros((nl,), jnp.float32)
            for t in range(k // _CHUNK):
                pltpu.sync_copy(zeros_v, shared.at[pl.ds(t * _CHUNK, _CHUNK)])

        plsc.subcore_barrier()

        for j in range(n_chunks):
            pltpu.sync_copy(ones_v, shared.at[idx_v.at[j]], add=True)

        for c in copies:
            c.wait()
        pltpu.sync_copy(rows_v, zq_hbm.at[pl.ds(wid * b_per_w, b_per_w)])

        plsc.subcore_barrier()

        @pl.when(sid == 0)
        def _dump():
            pltpu.sync_copy(shared, cnt_hbm.at[cid])

    return gather_hist


_sc_cache = {}


def _sc_gather_hist(embed, idx2):
    k = embed.shape[0]
    if k not in _sc_cache:
        _sc_cache[k] = _make_sc_gather_hist(k)
    return _sc_cache[k](embed, idx2)


def _perp_body(cl_ref, ch_ref, pl_ref, ph_ref):
    for c_ref, p_ref in ((cl_ref, pl_ref), (ch_ref, ph_ref)):
        c = c_ref[...]
        tot = c[0:1, :] + c[1:2, :]
        p = tot * (1.0 / float(M))
        ent = -jnp.sum(p * jnp.log(p + 1e-10))
        p_ref[...] = jnp.full((1, 1), jnp.exp(ent), jnp.float32)


@jax.jit
def _perp_tc(cnt_l, cnt_h):
    pl_, ph_ = pl.pallas_call(
        _perp_body,
        out_shape=[
            jax.ShapeDtypeStruct((1, 1), jnp.float32),
            jax.ShapeDtypeStruct((1, 1), jnp.float32),
        ],
    )(cnt_l, cnt_h)
    return pl_[0, 0], ph_[0, 0]


@jax.jit
def _run(z_l, z_h, embed_l, embed_h):
    idx_l, loss_l = _vq_tc(z_l, embed_l, 1024)
    zq_l, cnt_l = _sc_gather_hist(embed_l, idx_l.reshape(M // 128, 128))
    idx_h, loss_h = _vq_tc(z_h, embed_h, 1024)
    zq_h, cnt_h = _sc_gather_hist(embed_h, idx_h.reshape(M // 128, 128))
    perp_l, perp_h = _perp_tc(cnt_l, cnt_h)
    return (zq_l.reshape(B, N, D), idx_l.reshape(B, N), loss_l, perp_l,
            zq_h.reshape(B, N, D), idx_h.reshape(B, N), loss_h, perp_h)


def kernel(z_l, z_h, embed_l, embed_h):
    return _run(z_l, z_h, embed_l, embed_h)
